# baseline (device time: 50157 ns/iter reference)
import jax
import jax.numpy as jnp
from jax import lax
from jax.experimental import pallas as pl
from jax.experimental.pallas import tpu as pltpu

N_DEV = 16
SQ = 256
D = 1024
DH = 128
HQ_LOC = 8
SKV = 4096
CHUNK = SQ // N_DEV
SCALE = 0.08838834764831843
LOG2E = 1.4426950408889634
SCALE2 = SCALE * LOG2E


def _compute_body(x_ref, wq_ref, wo_ref, k_hbm, v_hbm, o_ref,
                  k_vmem, v_vmem, dma_sems):
    my = lax.axis_index("i")
    kv0 = 2 * my
    kcp = pltpu.make_async_copy(
        k_hbm.at[0, :, pl.ds(kv0, 2), :], k_vmem, dma_sems.at[0]
    )
    vcp = pltpu.make_async_copy(
        v_hbm.at[0, :, pl.ds(kv0, 2), :], v_vmem, dma_sems.at[1]
    )
    kcp.start()
    vcp.start()

    wob = wo_ref[...].astype(jnp.bfloat16)
    q = jnp.dot(x_ref[...], wq_ref[...], preferred_element_type=jnp.float32)
    qb = (q * SCALE2).astype(jnp.bfloat16)

    kcp.wait()
    vcp.wait()
    outs = []
    for g in range(2):
        k = k_vmem[:, g, :].astype(jnp.bfloat16)
        v = v_vmem[:, g, :].astype(jnp.bfloat16)
        qg = jnp.concatenate(
            [qb[:, (4 * g + hh) * DH:(4 * g + hh + 1) * DH] for hh in range(4)],
            axis=0,
        )
        s = lax.dot_general(
            qg, k, (((1,), (1,)), ((), ())),
            preferred_element_type=jnp.float32,
        )
        p = jnp.exp2(s)
        l = jnp.sum(p, axis=-1, keepdims=True)
        o = jnp.dot(
            p.astype(jnp.bfloat16), v, preferred_element_type=jnp.float32
        ) / l
        for hh in range(4):
            outs.append(o[hh * SQ:(hh + 1) * SQ, :])
    oheads = jnp.concatenate(outs, axis=1).astype(jnp.bfloat16)
    o_ref[...] = jnp.dot(
        oheads, wob, preferred_element_type=jnp.float32
    ).astype(jnp.bfloat16)


def _allreduce_body(p_ref, o_ref, rsbuf, redbuf,
                    rs_send, rs_recv, ag_send, ag_recv):
    my = lax.axis_index("i")

    rs_sends = []
    for d in range(1, N_DEV):
        j = lax.rem(my + d, N_DEV)
        rdma = pltpu.make_async_remote_copy(
            src_ref=p_ref.at[pl.ds(j * CHUNK, CHUNK), :],
            dst_ref=rsbuf.at[d - 1],
            send_sem=rs_send.at[d - 1],
            recv_sem=rs_recv.at[d - 1],
            device_id=(j,),
            device_id_type=pl.DeviceIdType.MESH,
        )
        rdma.start()
        rs_sends.append(rdma)

    def _wait(d):
        pltpu.make_async_remote_copy(
            src_ref=rsbuf.at[d - 1],
            dst_ref=rsbuf.at[d - 1],
            send_sem=rs_send.at[d - 1],
            recv_sem=rs_recv.at[d - 1],
            device_id=(my,),
            device_id_type=pl.DeviceIdType.MESH,
        ).wait_recv()

    for d in range(1, 9):
        _wait(d)
    acc = (
        p_ref[pl.ds(my * CHUNK, CHUNK), :].astype(jnp.float32)
        + jnp.sum(rsbuf[0:8].astype(jnp.float32), axis=0)
    )
    for d in range(9, N_DEV):
        _wait(d)
    red = (
        acc + jnp.sum(rsbuf[8:].astype(jnp.float32), axis=0)
    ).astype(jnp.bfloat16)
    redbuf[...] = red
    o_ref[pl.ds(my * CHUNK, CHUNK), :] = red

    ag_sends = []
    for d in range(1, N_DEV):
        j = lax.rem(my + d, N_DEV)
        rdma = pltpu.make_async_remote_copy(
            src_ref=redbuf,
            dst_ref=o_ref.at[pl.ds(my * CHUNK, CHUNK), :],
            send_sem=ag_send.at[d - 1],
            recv_sem=ag_recv.at[d - 1],
            device_id=(j,),
            device_id_type=pl.DeviceIdType.MESH,
        )
        rdma.start()
        ag_sends.append(rdma)

    for d in range(1, N_DEV):
        src = lax.rem(my - d + N_DEV, N_DEV)
        pltpu.make_async_remote_copy(
            src_ref=redbuf,
            dst_ref=o_ref.at[pl.ds(src * CHUNK, CHUNK), :],
            send_sem=ag_send.at[d - 1],
            recv_sem=ag_recv.at[d - 1],
            device_id=(my,),
            device_id_type=pl.DeviceIdType.MESH,
        ).wait_recv()

    for rdma in rs_sends + ag_sends:
        rdma.wait_send()


def kernel(x, Wq, Wo, K_ext, V_ext):
    partial = pl.pallas_call(
        _compute_body,
        out_shape=jax.ShapeDtypeStruct((SQ, D), jnp.bfloat16),
        in_specs=[
            pl.BlockSpec(memory_space=pltpu.VMEM),
            pl.BlockSpec(memory_space=pltpu.VMEM),
            pl.BlockSpec(memory_space=pltpu.VMEM),
            pl.BlockSpec(memory_space=pl.ANY),
            pl.BlockSpec(memory_space=pl.ANY),
        ],
        out_specs=pl.BlockSpec(memory_space=pltpu.VMEM),
        scratch_shapes=[
            pltpu.VMEM((SKV, 2, DH), jnp.float32),
            pltpu.VMEM((SKV, 2, DH), jnp.float32),
            pltpu.SemaphoreType.DMA((2,)),
        ],
    )(
        x[0],
        Wq,
        Wo,
        K_ext,
        V_ext,
    )

    out = pl.pallas_call(
        _allreduce_body,
        out_shape=jax.ShapeDtypeStruct((SQ, D), jnp.bfloat16),
        in_specs=[pl.BlockSpec(memory_space=pltpu.VMEM)],
        out_specs=pl.BlockSpec(memory_space=pltpu.VMEM),
        scratch_shapes=[
            pltpu.VMEM((N_DEV - 1, CHUNK, D), jnp.bfloat16),
            pltpu.VMEM((CHUNK, D), jnp.bfloat16),
            pltpu.SemaphoreType.DMA((N_DEV - 1,)),
            pltpu.SemaphoreType.DMA((N_DEV - 1,)),
            pltpu.SemaphoreType.DMA((N_DEV - 1,)),
            pltpu.SemaphoreType.DMA((N_DEV - 1,)),
        ],
    )(partial)

    return out[None].astype(jnp.float32)


# device time: 48326 ns/iter; 1.0379x vs baseline; 1.0379x over previous
import jax
import jax.numpy as jnp
from jax import lax
from jax.experimental import pallas as pl
from jax.experimental.pallas import tpu as pltpu

N_DEV = 16
SQ = 256
D = 1024
DH = 128
HQ_LOC = 8
SKV = 4096
CHUNK = SQ // N_DEV
SCALE = 0.08838834764831843
LOG2E = 1.4426950408889634
SCALE2 = SCALE * LOG2E


def _compute_body(x_ref, wq_ref, wo_ref, k_hbm, v_hbm, o_ref,
                  k_vmem, v_vmem, dma_sems):
    my = lax.axis_index("i")
    kv0 = 2 * my
    kcp = pltpu.make_async_copy(
        k_hbm.at[0, :, pl.ds(kv0, 2), :], k_vmem, dma_sems.at[0]
    )
    vcp = pltpu.make_async_copy(
        v_hbm.at[0, :, pl.ds(kv0, 2), :], v_vmem, dma_sems.at[1]
    )
    kcp.start()
    vcp.start()

    q = jnp.dot(x_ref[...], wq_ref[...], preferred_element_type=jnp.float32)
    qb = (q * SCALE2).astype(jnp.bfloat16)

    kcp.wait()
    vcp.wait()
    outs = []
    for g in range(2):
        k = k_vmem[:, g, :].astype(jnp.bfloat16)
        v = v_vmem[:, g, :].astype(jnp.bfloat16)
        qg = jnp.concatenate(
            [qb[:, (4 * g + hh) * DH:(4 * g + hh + 1) * DH] for hh in range(4)],
            axis=0,
        )
        s = lax.dot_general(
            qg, k, (((1,), (1,)), ((), ())),
            preferred_element_type=jnp.float32,
        )
        p = jnp.exp2(s)
        l = jnp.sum(p, axis=-1, keepdims=True)
        o = jnp.dot(
            p.astype(jnp.bfloat16), v, preferred_element_type=jnp.float32
        ) / l
        for hh in range(4):
            outs.append(o[hh * SQ:(hh + 1) * SQ, :])
    oheads = jnp.concatenate(outs, axis=1).astype(jnp.bfloat16)
    o_ref[...] = jnp.dot(
        oheads, wo_ref[...], preferred_element_type=jnp.float32
    ).astype(jnp.bfloat16)


def _allreduce_body(p_ref, o_ref, rsbuf, redbuf,
                    rs_send, rs_recv, ag_send, ag_recv):
    my = lax.axis_index("i")

    rs_sends = []
    for d in range(1, N_DEV):
        j = lax.rem(my + d, N_DEV)
        rdma = pltpu.make_async_remote_copy(
            src_ref=p_ref.at[pl.ds(j * CHUNK, CHUNK), :],
            dst_ref=rsbuf.at[d - 1],
            send_sem=rs_send.at[d - 1],
            recv_sem=rs_recv.at[d - 1],
            device_id=(j,),
            device_id_type=pl.DeviceIdType.MESH,
        )
        rdma.start()
        rs_sends.append(rdma)

    def _wait(d):
        pltpu.make_async_remote_copy(
            src_ref=rsbuf.at[d - 1],
            dst_ref=rsbuf.at[d - 1],
            send_sem=rs_send.at[d - 1],
            recv_sem=rs_recv.at[d - 1],
            device_id=(my,),
            device_id_type=pl.DeviceIdType.MESH,
        ).wait_recv()

    for d in range(1, 9):
        _wait(d)
    acc = (
        p_ref[pl.ds(my * CHUNK, CHUNK), :].astype(jnp.float32)
        + jnp.sum(rsbuf[0:8].astype(jnp.float32), axis=0)
    )
    for d in range(9, N_DEV):
        _wait(d)
    red = (
        acc + jnp.sum(rsbuf[8:].astype(jnp.float32), axis=0)
    ).astype(jnp.bfloat16)
    redbuf[...] = red
    o_ref[pl.ds(my * CHUNK, CHUNK), :] = red

    ag_sends = []
    for d in range(1, N_DEV):
        j = lax.rem(my + d, N_DEV)
        rdma = pltpu.make_async_remote_copy(
            src_ref=redbuf,
            dst_ref=o_ref.at[pl.ds(my * CHUNK, CHUNK), :],
            send_sem=ag_send.at[d - 1],
            recv_sem=ag_recv.at[d - 1],
            device_id=(j,),
            device_id_type=pl.DeviceIdType.MESH,
        )
        rdma.start()
        ag_sends.append(rdma)

    for d in range(1, N_DEV):
        src = lax.rem(my - d + N_DEV, N_DEV)
        pltpu.make_async_remote_copy(
            src_ref=redbuf,
            dst_ref=o_ref.at[pl.ds(src * CHUNK, CHUNK), :],
            send_sem=ag_send.at[d - 1],
            recv_sem=ag_recv.at[d - 1],
            device_id=(my,),
            device_id_type=pl.DeviceIdType.MESH,
        ).wait_recv()

    for rdma in rs_sends + ag_sends:
        rdma.wait_send()


def kernel(x, Wq, Wo, K_ext, V_ext):
    partial = pl.pallas_call(
        _compute_body,
        out_shape=jax.ShapeDtypeStruct((SQ, D), jnp.bfloat16),
        in_specs=[
            pl.BlockSpec(memory_space=pltpu.VMEM),
            pl.BlockSpec(memory_space=pltpu.VMEM),
            pl.BlockSpec(memory_space=pltpu.VMEM),
            pl.BlockSpec(memory_space=pl.ANY),
            pl.BlockSpec(memory_space=pl.ANY),
        ],
        out_specs=pl.BlockSpec(memory_space=pltpu.VMEM),
        scratch_shapes=[
            pltpu.VMEM((SKV, 2, DH), jnp.float32),
            pltpu.VMEM((SKV, 2, DH), jnp.float32),
            pltpu.SemaphoreType.DMA((2,)),
        ],
    )(
        x[0],
        Wq,
        Wo.astype(jnp.bfloat16),
        K_ext,
        V_ext,
    )

    out = pl.pallas_call(
        _allreduce_body,
        out_shape=jax.ShapeDtypeStruct((SQ, D), jnp.bfloat16),
        in_specs=[pl.BlockSpec(memory_space=pltpu.VMEM)],
        out_specs=pl.BlockSpec(memory_space=pltpu.VMEM),
        scratch_shapes=[
            pltpu.VMEM((N_DEV - 1, CHUNK, D), jnp.bfloat16),
            pltpu.VMEM((CHUNK, D), jnp.bfloat16),
            pltpu.SemaphoreType.DMA((N_DEV - 1,)),
            pltpu.SemaphoreType.DMA((N_DEV - 1,)),
            pltpu.SemaphoreType.DMA((N_DEV - 1,)),
            pltpu.SemaphoreType.DMA((N_DEV - 1,)),
        ],
    )(partial)

    return out[None].astype(jnp.float32)
